# Initial kernel scaffold; baseline (speedup 1.0000x reference)
import jax
import jax.numpy as jnp
from jax.experimental import pallas as pl


def kernel(s_p, v_p, s_w, v_w, pos_p, pos_w, edge_index_pw, edge_index_ww, params):
    raise NotImplementedError("write your pallas kernel here")


# speedup vs baseline: 8.2049x; 8.2049x over previous
import functools

import jax
import jax.numpy as jnp
from jax import lax
from jax.experimental import pallas as pl
from jax.experimental.pallas import tpu as pltpu
from jax.experimental.pallas import tpu_sc as plsc

F32 = jnp.float32
I32 = jnp.int32
NC = 2
NS = 16
RBF = 16
EPS = 1e-8


def _dot(a, b):
    return jnp.dot(a, b, preferred_element_type=F32)


def _gvp_tail(s, V, Wh, Ws, bs, Wu, Wg, bg):
    Vh = [_dot(V[c], Wh) for c in range(3)]
    vn = jnp.sqrt(Vh[0] * Vh[0] + Vh[1] * Vh[1] + Vh[2] * Vh[2] + EPS)
    so = _dot(s, Ws[0:128]) + _dot(vn, Ws[128:144]) + bs
    s2 = jnp.maximum(so, 0.0)
    gate = jax.nn.sigmoid(_dot(s2, Wg) + bg)
    V2 = [_dot(Vh[c], Wu) * gate for c in range(3)]
    return s2, V2


def _msg_body(gs_s, gs_d, gv_s, gv_d,
              Wh1, Ws1, bs1, Wu1, Wg1, bg1,
              Wh2, Ws2, bs2, Wu2, Wg2, bg2,
              Wh3, Ws3, bs3, Wu3, Wg3, bg3,
              sm_o, vm_o):
    ss = gs_s[...]
    sd = gs_d[...]
    vs = gv_s[...]
    vd = gv_d[...]
    d4 = vd[:, 48:52] - vs[:, 48:52]
    dist2 = jnp.sum(d4 * d4, axis=1, keepdims=True)
    dist = jnp.sqrt(dist2 + EPS)
    inv = 1.0 / dist
    mu = lax.broadcasted_iota(I32, (1, RBF), 1).astype(F32) * (20.0 / (RBF - 1))
    sig = 20.0 / RBF
    rbf = jnp.exp(-((dist - mu) ** 2) * (1.0 / (2.0 * sig * sig)))

    W1 = Wh1[...]
    Vh = []
    for c in range(3):
        du = d4[:, c:c + 1] * inv
        Vh.append(_dot(vs[:, 16 * c:16 * c + 16], W1[0:16])
                  + _dot(vd[:, 16 * c:16 * c + 16], W1[16:32])
                  + du * W1[32:33])
    vn = jnp.sqrt(Vh[0] * Vh[0] + Vh[1] * Vh[1] + Vh[2] * Vh[2] + EPS)
    so = (_dot(ss, Ws1[0:128]) + _dot(sd, Ws1[128:256])
          + _dot(rbf, Ws1[256:272]) + _dot(vn, Ws1[272:305]) + bs1[...])
    s = jnp.maximum(so, 0.0)
    gate = jax.nn.sigmoid(_dot(s, Wg1[...]) + bg1[...])
    V = [_dot(Vh[c], Wu1[...]) * gate for c in range(3)]

    s, V = _gvp_tail(s, V, Wh2[...], Ws2[...], bs2[...], Wu2[...], Wg2[...], bg2[...])
    s, V = _gvp_tail(s, V, Wh3[...], Ws3[...], bs3[...], Wu3[...], Wg3[...], bg3[...])

    sm_o[...] = s
    vm_o[:, 0:16] = V[0]
    vm_o[:, 16:32] = V[1]
    vm_o[:, 32:48] = V[2]
    vm_o[:, 48:128] = jnp.zeros((s.shape[0], 80), F32)


def _upd_body(sw, aS, vw, aV,
              Wh1, Ws1, bs1, Wu1, Wg1, bg1,
              Wh2, Ws2, bs2, Wu2, Wg2, bg2,
              Wh3, Ws3, bs3, Wu3, Wg3, bg3,
              s_o, v_o):
    sw_v = sw[...]
    vw_v = vw[...]
    av = aV[...]
    W1 = Wh1[...]
    Vh = []
    for c in range(3):
        Vh.append(_dot(vw_v[:, 16 * c:16 * c + 16], W1[0:16])
                  + _dot(av[:, 16 * c:16 * c + 16], W1[16:32]))
    vn = jnp.sqrt(Vh[0] * Vh[0] + Vh[1] * Vh[1] + Vh[2] * Vh[2] + EPS)
    so = (_dot(sw_v, Ws1[0:128]) + _dot(aS[...], Ws1[128:256])
          + _dot(vn, Ws1[256:288]) + bs1[...])
    s = jnp.maximum(so, 0.0)
    gate = jax.nn.sigmoid(_dot(s, Wg1[...]) + bg1[...])
    V = [_dot(Vh[c], Wu1[...]) * gate for c in range(3)]

    s, V = _gvp_tail(s, V, Wh2[...], Ws2[...], bs2[...], Wu2[...], Wg2[...], bg2[...])
    s, V = _gvp_tail(s, V, Wh3[...], Ws3[...], bs3[...], Wu3[...], Wg3[...], bg3[...])

    s_o[...] = sw_v + s
    v_o[:, 0:16] = vw_v[:, 0:16] + V[0]
    v_o[:, 16:32] = vw_v[:, 16:32] + V[1]
    v_o[:, 32:48] = vw_v[:, 32:48] + V[2]


def _stack_args(stk):
    out = []
    for p in stk:
        out += [p['Wh'], p['Ws'], p['bs'].reshape(1, -1),
                p['Wu'], p['Wg'], p['bg'].reshape(1, -1)]
    return out


def _full_spec(shape):
    return pl.BlockSpec(shape, lambda i: (0, 0))


def _row_spec(blk, cols):
    return pl.BlockSpec((blk, cols), lambda i: (i, 0))


def _msg_call(gs_s, gs_d, gv_s, gv_d, stk, e_pad):
    blk = 2048
    wargs = _stack_args(stk)
    in_specs = [_row_spec(blk, 128), _row_spec(blk, 128),
                _row_spec(blk, 128), _row_spec(blk, 128)]
    in_specs += [_full_spec(w.shape) for w in wargs]
    out_specs = [_row_spec(blk, 128)] * 2
    out_shape = [jax.ShapeDtypeStruct((e_pad, 128), F32)] * 2
    return pl.pallas_call(
        _msg_body,
        grid=(e_pad // blk,),
        in_specs=in_specs,
        out_specs=out_specs,
        out_shape=out_shape,
        compiler_params=pltpu.CompilerParams(
            dimension_semantics=("arbitrary",)),
    )(gs_s, gs_d, gv_s, gv_d, *wargs)


def _upd_call(s_w, aggS, vw48, aggV, stk, n_w):
    blk = 2000
    wargs = _stack_args(stk)
    in_specs = [_row_spec(blk, 128), _row_spec(blk, 128),
                _row_spec(blk, 48), _row_spec(blk, 128)]
    in_specs += [_full_spec(w.shape) for w in wargs]
    out_specs = [_row_spec(blk, 128), _row_spec(blk, 48)]
    out_shape = [jax.ShapeDtypeStruct((n_w, 128), F32),
                 jax.ShapeDtypeStruct((n_w, 48), F32)]
    return pl.pallas_call(
        _upd_body,
        grid=(n_w // blk,),
        in_specs=in_specs,
        out_specs=out_specs,
        out_shape=out_shape,
        compiler_params=pltpu.CompilerParams(
            dimension_semantics=("arbitrary",)),
    )(s_w, aggS, vw48, aggV, *wargs)


def _gather_call(s_tab, vp_tab, idx_list, e_pad):
    n_idx = len(idx_list)
    n_chunks = e_pad // 128
    cpt = n_chunks // (NC * NS)
    mesh = plsc.VectorSubcoreMesh(core_axis_name="c", subcore_axis_name="s")
    out_type = []
    for _ in range(n_idx):
        out_type += [jax.ShapeDtypeStruct((e_pad, 128), F32),
                     jax.ShapeDtypeStruct((e_pad, 128), F32)]
    scratch = [pltpu.VMEM((128,), I32),
               pltpu.VMEM((128, 128), F32),
               pltpu.VMEM((128, 128), F32),
               pltpu.SemaphoreType.DMA,
               pltpu.SemaphoreType.DMA]

    @functools.partial(pl.kernel, out_type=tuple(out_type), mesh=mesh,
                       scratch_types=scratch)
    def k(*refs):
        stab = refs[0]
        vtab = refs[1]
        idxs = refs[2:2 + n_idx]
        outs = refs[2 + n_idx:2 + 3 * n_idx]
        idxrow, sbuf, vbuf, sem1, sem2 = refs[2 + 3 * n_idx:]
        cid = lax.axis_index("c")
        sid = lax.axis_index("s")
        w = sid * NC + cid
        for q in range(n_idx):
            idx_hbm = idxs[q]
            out_s = outs[2 * q]
            out_v = outs[2 * q + 1]

            def body(j, _, idx_hbm=idx_hbm, out_s=out_s, out_v=out_v):
                row = w * cpt + j
                pltpu.sync_copy(idx_hbm.at[row], idxrow)
                cp1 = pltpu.async_copy(stab.at[idxrow], sbuf, sem1)
                cp2 = pltpu.async_copy(vtab.at[idxrow], vbuf, sem2)
                cp1.wait()
                cp2.wait()
                pltpu.sync_copy(sbuf, out_s.at[pl.ds(row * 128, 128)])
                pltpu.sync_copy(vbuf, out_v.at[pl.ds(row * 128, 128)])
                return 0

            lax.fori_loop(0, cpt, body, 0)

    return k(s_tab, vp_tab, *idx_list)


def _scatter_call(dst_a, dst_b, msgs_a, msgs_b, zeros_z, e, e_pad, n_w, cap):
    n_chunks = e_pad // 128
    cpt = n_chunks // NS
    prt = cap // NS
    mesh = plsc.VectorSubcoreMesh(core_axis_name="c", subcore_axis_name="s")
    out_type = tuple(jax.ShapeDtypeStruct((4 * cap, 128), F32) for _ in range(2))
    scratch = [pltpu.VMEM_SHARED((cap + 8, 128), F32),
               pltpu.VMEM((128,), I32),
               pltpu.VMEM((1, 128), I32),
               pltpu.VMEM((128, 128), F32)]

    @functools.partial(pl.kernel, out_type=out_type, mesh=mesh,
                       scratch_types=scratch)
    def k(dsta, dstb, mS_a, mV_a, mS_b, mV_b, z_hbm,
          outS, outV, acc, idxrow, lidx, mbuf):
        cid = lax.axis_index("c")
        sid = lax.axis_index("s")
        groups = ((outS, mS_a, mS_b), (outV, mV_a, mV_b))
        zfull = prt // 128
        zrem = prt - zfull * 128
        for out, m_a, m_b in groups:
            for p in range(2):
                base = (2 * p + cid) * cap

                pltpu.sync_copy(z_hbm, mbuf)

                def zbody(i, _):
                    pltpu.sync_copy(mbuf, acc.at[pl.ds(sid * prt + i * 128, 128)])
                    return 0
                lax.fori_loop(0, zfull, zbody, 0)
                if zrem:
                    pltpu.sync_copy(
                        mbuf.at[pl.ds(0, zrem)],
                        acc.at[pl.ds(sid * prt + zfull * 128, zrem)])
                plsc.subcore_barrier()
                for dst_hbm, msg in ((dsta, m_a), (dstb, m_b)):
                    def body(j, _, dst_hbm=dst_hbm, msg=msg, base=base):
                        row = sid * cpt + j
                        pltpu.sync_copy(dst_hbm.at[row], idxrow)
                        for t in range(8):
                            v = idxrow[pl.ds(16 * t, 16)]
                            gpos = (row * 128 + 16 * t
                                    + lax.broadcasted_iota(I32, (16,), 0))
                            ok = (gpos < e) & (v >= base) & (v < base + cap)
                            lidx[0, pl.ds(16 * t, 16)] = jnp.where(ok, v - base, cap)
                        pltpu.sync_copy(msg.at[pl.ds(row * 128, 128)], mbuf)
                        pltpu.sync_copy(mbuf, acc.at[lidx.at[0]], add=True)
                        return 0
                    lax.fori_loop(0, cpt, body, 0)
                plsc.subcore_barrier()
                pltpu.sync_copy(acc.at[pl.ds(sid * prt, prt)],
                                out.at[pl.ds(base + sid * prt, prt)])
                plsc.subcore_barrier()

    return k(dst_a, dst_b, msgs_a[0], msgs_a[1], msgs_b[0], msgs_b[1], zeros_z)


def _pad_idx(idx, e_pad):
    e = idx.shape[0]
    idx = jnp.pad(idx.astype(I32), (0, e_pad - e))
    return idx.reshape(e_pad // 128, 128)


def kernel(s_p, v_p, s_w, v_w, pos_p, pos_w, edge_index_pw, edge_index_ww,
           params):
    n_p = s_p.shape[0]
    n_w = s_w.shape[0]
    e = edge_index_pw.shape[1]
    e_pad = ((e + 4095) // 4096) * 4096
    cap = ((n_w // 4 + 3199) // 3200) * 3200

    vp_p = jnp.concatenate([
        v_p.transpose(0, 2, 1).reshape(n_p, 48), pos_p,
        jnp.zeros((n_p, 77), F32)], axis=1)
    vw48 = v_w.transpose(0, 2, 1).reshape(n_w, 48)
    src_pw = _pad_idx(edge_index_pw[0], e_pad)
    dst_pw = _pad_idx(edge_index_pw[1], e_pad)
    src_ww = _pad_idx(edge_index_ww[0], e_pad)
    dst_ww = _pad_idx(edge_index_ww[1], e_pad)
    zeros_z = jnp.zeros((128, 128), F32)
    posw_pad = jnp.concatenate([pos_w, jnp.zeros((n_w, 77), F32)], axis=1)

    gs_spw, gvp_spw = _gather_call(s_p, vp_p, [src_pw], e_pad)

    sw = s_w
    vw = vw48
    for blk in params['blocks']:
        vp_w = jnp.concatenate([vw, posw_pad], axis=1)
        g = _gather_call(sw, vp_w, [dst_pw, src_ww, dst_ww], e_pad)
        gs_dpw, gvp_dpw, gs_sww, gvp_sww, gs_dww, gvp_dww = g

        mS_pw, mV_pw = _msg_call(
            gs_spw, gs_dpw, gvp_spw, gvp_dpw, blk['pw'], e_pad)
        mS_ww, mV_ww = _msg_call(
            gs_sww, gs_dww, gvp_sww, gvp_dww, blk['ww'], e_pad)

        aggS, aggV = _scatter_call(
            dst_pw, dst_ww, (mS_pw, mV_pw), (mS_ww, mV_ww),
            zeros_z, e, e_pad, n_w, cap)

        sw, vw = _upd_call(sw, aggS, vw, aggV, blk['upd'], n_w)

    v_w_out = vw.reshape(n_w, 3, 16).transpose(0, 2, 1)
    return (s_p, v_p, sw, v_w_out)
